# Initial kernel scaffold; baseline (speedup 1.0000x reference)
import jax
import jax.numpy as jnp
from jax.experimental import pallas as pl


def kernel(x, edge_index, spatial_list, wnn_list, W1, a_src1, a_dst1, b1, W2, a_src2, a_dst2, b2, disc_W, disc_b):
    raise NotImplementedError("write your pallas kernel here")


# speedup vs baseline: 4.1997x; 4.1997x over previous
import functools

import jax
import jax.numpy as jnp
from jax import lax
from jax.experimental import pallas as pl
from jax.experimental.pallas import tpu as pltpu
from jax.experimental.pallas import tpu_sc as plsc

_N = 10000
_NP = 10240
_D = 256
_H = 128
_BN = 1280
_GRID = _NP // _BN
_TPB = 640


def _mm_alphas_body(x, w_ref, as_ref, ad_ref, h_ref, s_ref, d_ref, m_ref):
    h = jnp.dot(x, w_ref[...], preferred_element_type=jnp.float32)
    h_ref[...] = jnp.stack([h[:, :_H], h[:, _H:]])
    s = jnp.dot(h, as_ref[...], preferred_element_type=jnp.float32)
    d = jnp.dot(h, ad_ref[...], preferred_element_type=jnp.float32)
    s_ref[...] = s
    d_ref[...] = d
    cur = jnp.concatenate([jnp.max(s).reshape(1, 1), jnp.max(d).reshape(1, 1)],
                          axis=1)
    i = pl.program_id(0)

    @pl.when(i == 0)
    def _():
        m_ref[...] = cur

    @pl.when(i > 0)
    def _():
        m_ref[...] = jnp.maximum(m_ref[...], cur)


def _k1_body(x_ref, w_ref, as_ref, ad_ref, h_ref, s_ref, d_ref, m_ref):
    _mm_alphas_body(x_ref[...], w_ref, as_ref, ad_ref, h_ref, s_ref, d_ref,
                    m_ref)


def _k2_body(a_ref, den_ref, b_ref, w_ref, as_ref, ad_ref,
             h_ref, s_ref, d_ref, m_ref):
    acc = a_ref[...]
    xcat = jnp.concatenate([acc[0], acc[1]], axis=1)
    den = den_ref[...]
    dsum = den[:, 0:1] + den[:, 1:2]
    xin = jnp.where(dsum > 0, xcat / dsum, 0.0) + b_ref[...]
    xin = jnp.maximum(xin, 0.0)
    _mm_alphas_body(xin, w_ref, as_ref, ad_ref, h_ref, s_ref, d_ref, m_ref)


def _kemb_body(a_ref, den_ref, b_ref, h2_ref, e_ref):
    acc = a_ref[...]
    xcat = jnp.concatenate([acc[0], acc[1]], axis=1)
    den = den_ref[...]
    dsum = den[:, 0:1] + den[:, 1:2]
    h2 = xcat / dsum + b_ref[...]
    h2_ref[...] = h2
    emb = jnp.maximum(h2, 0.0)
    e_ref[...] = jnp.stack([emb[:, :_H], emb[:, _H:]])


def _k3_body(ar_ref, dr_ref, e_ref, a2_ref, d2_ref, b_ref, w_ref, db_ref,
             out_ref):
    ar = ar_ref[...]
    gpre = jnp.concatenate([ar[0], ar[1]], axis=1)
    dr = dr_ref[...]
    cnt = dr[:, 0:1] + dr[:, 1:2]
    gpre = gpre / cnt
    nrm = jnp.sqrt(jnp.sum(gpre * gpre, axis=1, keepdims=True))
    nrm = jnp.maximum(nrm, 1e-12)
    g = jax.nn.sigmoid(gpre / nrm)

    e = e_ref[...]
    emb = jnp.concatenate([e[0], e[1]], axis=1)
    a2 = a2_ref[...]
    d2 = d2_ref[...]
    dsum2 = d2[:, 0:1] + d2[:, 1:2]
    emba = jnp.maximum(jnp.concatenate([a2[0], a2[1]], axis=1) / dsum2
                       + b_ref[...], 0.0)

    w = w_ref[...]
    db = db_ref[...]
    e1 = jnp.dot(emb, w, preferred_element_type=jnp.float32)
    sc1 = jnp.sum(e1 * g, axis=1, keepdims=True) + db
    e2 = jnp.dot(emba, w, preferred_element_type=jnp.float32)
    sc2 = jnp.sum(e2 * g, axis=1, keepdims=True) + db
    out_ref[...] = jnp.concatenate([sc1, sc2], axis=1)


def _run_k1(x, w, a_s, a_d):
    return pl.pallas_call(
        _k1_body,
        grid=(_GRID,),
        in_specs=[
            pl.BlockSpec((_BN, _D), lambda i: (i, 0)),
            pl.BlockSpec((_D, _D), lambda i: (0, 0)),
            pl.BlockSpec((_D, 1), lambda i: (0, 0)),
            pl.BlockSpec((_D, 1), lambda i: (0, 0)),
        ],
        out_specs=[
            pl.BlockSpec((2, _BN, _H), lambda i: (0, i, 0)),
            pl.BlockSpec((_BN, 1), lambda i: (i, 0)),
            pl.BlockSpec((_BN, 1), lambda i: (i, 0)),
            pl.BlockSpec((1, 2), lambda i: (0, 0)),
        ],
        out_shape=[
            jax.ShapeDtypeStruct((2, _NP, _H), jnp.float32),
            jax.ShapeDtypeStruct((_NP, 1), jnp.float32),
            jax.ShapeDtypeStruct((_NP, 1), jnp.float32),
            jax.ShapeDtypeStruct((1, 2), jnp.float32),
        ],
    )(x, w, a_s, a_d)


def _run_k2(acc, dent, b, w, a_s, a_d):
    return pl.pallas_call(
        _k2_body,
        grid=(_GRID,),
        in_specs=[
            pl.BlockSpec((2, _BN, _H), lambda i: (0, i, 0)),
            pl.BlockSpec((_BN, 2), lambda i: (i, 0)),
            pl.BlockSpec((1, _D), lambda i: (0, 0)),
            pl.BlockSpec((_D, _D), lambda i: (0, 0)),
            pl.BlockSpec((_D, 1), lambda i: (0, 0)),
            pl.BlockSpec((_D, 1), lambda i: (0, 0)),
        ],
        out_specs=[
            pl.BlockSpec((2, _BN, _H), lambda i: (0, i, 0)),
            pl.BlockSpec((_BN, 1), lambda i: (i, 0)),
            pl.BlockSpec((_BN, 1), lambda i: (i, 0)),
            pl.BlockSpec((1, 2), lambda i: (0, 0)),
        ],
        out_shape=[
            jax.ShapeDtypeStruct((2, _NP, _H), jnp.float32),
            jax.ShapeDtypeStruct((_NP, 1), jnp.float32),
            jax.ShapeDtypeStruct((_NP, 1), jnp.float32),
            jax.ShapeDtypeStruct((1, 2), jnp.float32),
        ],
    )(acc, dent, b, w, a_s, a_d)


def _run_kemb(acc, dent, b):
    return pl.pallas_call(
        _kemb_body,
        grid=(_GRID,),
        in_specs=[
            pl.BlockSpec((2, _BN, _H), lambda i: (0, i, 0)),
            pl.BlockSpec((_BN, 2), lambda i: (i, 0)),
            pl.BlockSpec((1, _D), lambda i: (0, 0)),
        ],
        out_specs=[
            pl.BlockSpec((_BN, _D), lambda i: (i, 0)),
            pl.BlockSpec((2, _BN, _H), lambda i: (0, i, 0)),
        ],
        out_shape=[
            jax.ShapeDtypeStruct((_NP, _D), jnp.float32),
            jax.ShapeDtypeStruct((2, _NP, _H), jnp.float32),
        ],
    )(acc, dent, b)


def _run_k3(accr, denrt, embt, acc2a, den2at, b2, disc_w, disc_b):
    return pl.pallas_call(
        _k3_body,
        grid=(_GRID,),
        in_specs=[
            pl.BlockSpec((2, _BN, _H), lambda i: (0, i, 0)),
            pl.BlockSpec((_BN, 2), lambda i: (i, 0)),
            pl.BlockSpec((2, _BN, _H), lambda i: (0, i, 0)),
            pl.BlockSpec((2, _BN, _H), lambda i: (0, i, 0)),
            pl.BlockSpec((_BN, 2), lambda i: (i, 0)),
            pl.BlockSpec((1, _D), lambda i: (0, 0)),
            pl.BlockSpec((_D, _D), lambda i: (0, 0)),
            pl.BlockSpec((1, 1), lambda i: (0, 0)),
        ],
        out_specs=pl.BlockSpec((_BN, 2), lambda i: (i, 0)),
        out_shape=jax.ShapeDtypeStruct((_NP, 2), jnp.float32),
    )(accr, denrt, embt, acc2a, den2at, b2, disc_w, disc_b)


@functools.lru_cache(maxsize=None)
def _make_scalar_pass(m_rows, e_true):
    rpw = m_rows // 32
    mesh = plsc.VectorSubcoreMesh(core_axis_name="c", subcore_axis_name="s")

    @functools.partial(
        pl.kernel,
        out_type=(
            jax.ShapeDtypeStruct((m_rows, 128), jnp.float32),
            jax.ShapeDtypeStruct((2, _NP), jnp.float32),
        ),
        mesh=mesh,
        compiler_params=pltpu.CompilerParams(needs_layout_passes=False),
        scratch_types=[
            pltpu.VMEM((_NP,), jnp.float32),
            pltpu.VMEM((_NP,), jnp.float32),
            pltpu.VMEM((16,), jnp.float32),
            pltpu.VMEM((128,), jnp.int32),
            pltpu.VMEM((128,), jnp.int32),
            pltpu.VMEM((128,), jnp.int32),
            pltpu.VMEM((128,), jnp.float32),
            pltpu.VMEM((_TPB,), jnp.float32),
            pltpu.VMEM_SHARED((_NP,), jnp.float32),
        ],
    )
    def scalar_pass(as_hbm, ad_hbm, sg_hbm, dg_hbm, ds_hbm, m_hbm,
                    ex_hbm, den_hbm,
                    as_v, ad_v, m_v, sgb, dgb, dsb, exb, zb, den_sp):
        c = lax.axis_index("c")
        s = lax.axis_index("s")
        w = s * 2 + c
        pltpu.sync_copy(as_hbm, as_v)
        pltpu.sync_copy(ad_hbm, ad_v)
        pltpu.sync_copy(m_hbm, m_v)

        z16 = jnp.zeros((16,), jnp.float32)

        def _zero(k, carry):
            zb[pl.ds(k * 16, 16)] = z16
            return carry

        lax.fori_loop(0, _TPB // 16, _zero, 0)
        pltpu.sync_copy(zb, den_sp.at[pl.ds(s * _TPB, _TPB)])
        plsc.subcore_barrier()

        mv = m_v[...]

        def _body(j, carry):
            jr = w * rpw + j
            pltpu.sync_copy(sg_hbm.at[jr], sgb)
            pltpu.sync_copy(dg_hbm.at[jr], dgb)
            pltpu.sync_copy(ds_hbm.at[jr], dsb)
            base = jr * 128
            for k in range(8):
                si = sgb[pl.ds(k * 16, 16)]
                di = dgb[pl.ds(k * 16, 16)]
                av = plsc.load_gather(as_v, [si])
                bv = plsc.load_gather(ad_v, [di])
                z = av + bv
                e = jnp.where(z >= 0, z, 0.2 * z)
                ex = jnp.exp(e - mv)
                eidx = base + k * 16 + lax.iota(jnp.int32, 16)
                ex = jnp.where(eidx < e_true, ex, 0.0)
                exb[pl.ds(k * 16, 16)] = ex
            pltpu.sync_copy(exb, ex_hbm.at[jr])
            pltpu.sync_copy(exb, den_sp.at[dsb], add=True)
            return carry

        lax.fori_loop(0, rpw, _body, 0)
        plsc.subcore_barrier()
        pltpu.sync_copy(den_sp.at[pl.ds(s * _TPB, _TPB)],
                        den_hbm.at[c, pl.ds(s * _TPB, _TPB)])

    return scalar_pass


@functools.lru_cache(maxsize=None)
def _make_row_pass(m_rows):
    rpw = m_rows // 16
    mesh = plsc.VectorSubcoreMesh(core_axis_name="c", subcore_axis_name="s")

    @functools.partial(
        pl.kernel,
        out_type=jax.ShapeDtypeStruct((2, _NP, _H), jnp.float32),
        mesh=mesh,
        compiler_params=pltpu.CompilerParams(needs_layout_passes=False),
        scratch_types=[
            pltpu.VMEM((128, _H), jnp.float32),
            pltpu.VMEM((128,), jnp.int32),
            pltpu.VMEM((128,), jnp.int32),
            pltpu.VMEM((128,), jnp.float32),
            pltpu.SemaphoreType.DMA,
            pltpu.VMEM_SHARED((_NP, _H), jnp.float32),
        ],
    )
    def row_pass(tab_hbm, sg_hbm, ds_hbm, ex_hbm, acc_hbm,
                 rows, sb, db, eb, sem, acc_sp):
        c = lax.axis_index("c")
        s = lax.axis_index("s")

        z16 = jnp.zeros((16,), jnp.float32)

        def _zrow(i, carry):
            for k in range(_H // 16):
                rows[i, pl.ds(k * 16, 16)] = z16
            return carry

        lax.fori_loop(0, 128, _zrow, 0)

        def _zcp(t, carry):
            pltpu.sync_copy(rows,
                            acc_sp.at[pl.ds(s * _TPB + t * 128, 128)])
            return carry

        lax.fori_loop(0, _TPB // 128, _zcp, 0)
        plsc.subcore_barrier()

        coff = c * _NP

        def _body(j, carry):
            jr = s * rpw + j
            pltpu.sync_copy(sg_hbm.at[jr], sb)
            pltpu.sync_copy(ds_hbm.at[jr], db)
            pltpu.sync_copy(ex_hbm.at[jr], eb)
            for k in range(8):
                sb[pl.ds(k * 16, 16)] = sb[pl.ds(k * 16, 16)] + coff
            pltpu.async_copy(tab_hbm.at[sb], rows, sem).wait()

            def _scale(i, icarry):
                wv = plsc.load_gather(eb, [jnp.zeros((16,), jnp.int32) + i])
                for k in range(_H // 16):
                    rows[i, pl.ds(k * 16, 16)] = (
                        rows[i, pl.ds(k * 16, 16)] * wv)
                return icarry

            lax.fori_loop(0, 128, _scale, 0)
            pltpu.sync_copy(rows, acc_sp.at[db], add=True)
            return carry

        lax.fori_loop(0, rpw, _body, 0)
        plsc.subcore_barrier()

        def _ocp(t, carry):
            r0 = s * _TPB + t * 128
            pltpu.sync_copy(acc_sp.at[pl.ds(r0, 128)],
                            acc_hbm.at[c, pl.ds(r0, 128)])
            return carry

        lax.fori_loop(0, _TPB // 128, _ocp, 0)

    return row_pass


def kernel(x, edge_index, spatial_list, wnn_list, W1, a_src1, a_dst1, b1,
           W2, a_src2, a_dst2, b2, disc_W, disc_b):
    i32 = jnp.int32
    f32 = jnp.float32
    n = x.shape[0]

    loops = jnp.arange(n, dtype=i32)
    perm = jax.random.permutation(jax.random.key(42), n).astype(i32)

    e1_true = edge_index.shape[1] + n
    ep1 = ((e1_true + 4095) // 4096) * 4096
    er_true = edge_index.shape[1]
    epr = ((er_true + 4095) // 4096) * 4096

    def pad2d(a, ep):
        a = a.astype(i32)
        a = jnp.concatenate([a, jnp.zeros((ep - a.shape[0],), i32)])
        return a.reshape(ep // 128, 128)

    src_w = jnp.concatenate([wnn_list[0].astype(i32), loops])
    dst_w = jnp.concatenate([wnn_list[1].astype(i32), loops])
    src_s = jnp.concatenate([spatial_list[0].astype(i32), loops])
    dst_s = jnp.concatenate([spatial_list[1].astype(i32), loops])

    sgw1 = pad2d(src_w, ep1)
    dgw1 = pad2d(dst_w, ep1)
    sgw2 = pad2d(perm[src_w], ep1)
    dgw2 = pad2d(perm[dst_w], ep1)
    sgs = pad2d(src_s, ep1)
    dgs = pad2d(dst_s, ep1)
    sgr = pad2d(edge_index[1], epr)
    dsr = pad2d(edge_index[0], epr)

    xp = jnp.pad(x.astype(f32), ((0, _NP - n), (0, 0)))
    as1_2d = a_src1.reshape(_D, 1).astype(f32)
    ad1_2d = a_dst1.reshape(_D, 1).astype(f32)
    as2_2d = a_src2.reshape(_D, 1).astype(f32)
    ad2_2d = a_dst2.reshape(_D, 1).astype(f32)
    b1_2d = b1.reshape(1, _D).astype(f32)
    b2_2d = b2.reshape(1, _D).astype(f32)
    db_2d = disc_b.reshape(1, 1).astype(f32)

    def mvec(m):
        return jnp.full((16,), jnp.maximum(m[0, 0] + m[0, 1], 0.0), f32)

    scalar_gat = _make_scalar_pass(ep1 // 128, e1_true)
    row_gat = _make_row_pass(ep1 // 128)
    scalar_r = _make_scalar_pass(epr // 128, er_true)
    row_r = _make_row_pass(epr // 128)

    h1t, as1, ad1, m1 = _run_k1(xp, W1.astype(f32), as1_2d, ad1_2d)
    tab1 = h1t.reshape(2 * _NP, _H)
    as1f = as1.reshape(_NP)
    ad1f = ad1.reshape(_NP)
    m1v = mvec(m1)

    ex11, den11 = scalar_gat(as1f, ad1f, sgw1, dgw1, dgw1, m1v)
    ex12, den12 = scalar_gat(as1f, ad1f, sgw2, dgw2, dgw1, m1v)
    acc11 = row_gat(tab1, sgw1, dgw1, ex11)
    acc12 = row_gat(tab1, sgw2, dgw1, ex12)

    h2t1, as21, ad21, m21 = _run_k2(acc11, den11.T, b1_2d, W2.astype(f32),
                                    as2_2d, ad2_2d)
    h2t2, as22, ad22, m22 = _run_k2(acc12, den12.T, b1_2d, W2.astype(f32),
                                    as2_2d, ad2_2d)

    ex21, den21 = scalar_gat(as21.reshape(_NP), ad21.reshape(_NP),
                             sgs, dgs, dgs, mvec(m21))
    ex22, den22 = scalar_gat(as22.reshape(_NP), ad22.reshape(_NP),
                             sgs, dgs, dgs, mvec(m22))
    acc21 = row_gat(h2t1.reshape(2 * _NP, _H), sgs, dgs, ex21)
    acc22 = row_gat(h2t2.reshape(2 * _NP, _H), sgs, dgs, ex22)

    h2_full, embt = _run_kemb(acc21, den21.T, b2_2d)

    zn = jnp.zeros((_NP,), f32)
    z16 = jnp.zeros((16,), f32)
    exr, denr = scalar_r(zn, zn, sgr, sgr, dsr, z16)
    accr = row_r(embt.reshape(2 * _NP, _H), sgr, dsr, exr)

    ret_full = _run_k3(accr, denr.T, embt, acc22, den22.T, b2_2d,
                       disc_W.astype(f32), db_2d)

    return h2_full[:n], ret_full[:n]


# speedup vs baseline: 5.9844x; 1.4250x over previous
import functools

import jax
import jax.numpy as jnp
from jax import lax
from jax.experimental import pallas as pl
from jax.experimental.pallas import tpu as pltpu
from jax.experimental.pallas import tpu_sc as plsc

_N = 10000
_NP = 10240
_D = 256
_H = 128
_BN = 1280
_GRID = _NP // _BN
_TPB = 640


def _mm_alphas_body(x, w_ref, as_ref, ad_ref, h_ref, s_ref, d_ref, m_ref):
    h = jnp.dot(x, w_ref[...], preferred_element_type=jnp.float32)
    h_ref[...] = jnp.stack([h[:, :_H], h[:, _H:]])
    s = jnp.dot(h, as_ref[...], preferred_element_type=jnp.float32)
    d = jnp.dot(h, ad_ref[...], preferred_element_type=jnp.float32)
    s_ref[...] = s
    d_ref[...] = d
    cur = jnp.concatenate([jnp.max(s).reshape(1, 1), jnp.max(d).reshape(1, 1)],
                          axis=1)
    i = pl.program_id(0)

    @pl.when(i == 0)
    def _():
        m_ref[...] = cur

    @pl.when(i > 0)
    def _():
        m_ref[...] = jnp.maximum(m_ref[...], cur)


def _k1_body(x_ref, w_ref, as_ref, ad_ref, h_ref, s_ref, d_ref, m_ref):
    _mm_alphas_body(x_ref[...], w_ref, as_ref, ad_ref, h_ref, s_ref, d_ref,
                    m_ref)


def _k2_body(a_ref, den_ref, b_ref, w_ref, as_ref, ad_ref,
             h_ref, s_ref, d_ref, m_ref):
    acc = a_ref[...]
    xcat = jnp.concatenate([acc[0], acc[1]], axis=1)
    dsum = den_ref[...]
    xin = jnp.where(dsum > 0, xcat / dsum, 0.0) + b_ref[...]
    xin = jnp.maximum(xin, 0.0)
    _mm_alphas_body(xin, w_ref, as_ref, ad_ref, h_ref, s_ref, d_ref, m_ref)


def _kemb_body(a_ref, den_ref, b_ref, h2_ref, e_ref):
    acc = a_ref[...]
    xcat = jnp.concatenate([acc[0], acc[1]], axis=1)
    h2 = xcat / den_ref[...] + b_ref[...]
    h2_ref[...] = h2
    emb = jnp.maximum(h2, 0.0)
    e_ref[...] = jnp.stack([emb[:, :_H], emb[:, _H:]])


def _k3_body(ar_ref, dr_ref, e_ref, a2_ref, d2_ref, b_ref, w_ref, db_ref,
             out_ref):
    ar = ar_ref[...]
    gpre = jnp.concatenate([ar[0], ar[1]], axis=1)
    gpre = gpre / dr_ref[...]
    nrm = jnp.sqrt(jnp.sum(gpre * gpre, axis=1, keepdims=True))
    nrm = jnp.maximum(nrm, 1e-12)
    g = jax.nn.sigmoid(gpre / nrm)

    e = e_ref[...]
    emb = jnp.concatenate([e[0], e[1]], axis=1)
    a2 = a2_ref[...]
    emba = jnp.maximum(
        jnp.concatenate([a2[0], a2[1]], axis=1) / d2_ref[...] + b_ref[...],
        0.0)

    w = w_ref[...]
    db = db_ref[...]
    e1 = jnp.dot(emb, w, preferred_element_type=jnp.float32)
    sc1 = jnp.sum(e1 * g, axis=1, keepdims=True) + db
    e2 = jnp.dot(emba, w, preferred_element_type=jnp.float32)
    sc2 = jnp.sum(e2 * g, axis=1, keepdims=True) + db
    out_ref[...] = jnp.concatenate([sc1, sc2], axis=1)


def _run_k1(x, w, a_s, a_d):
    return pl.pallas_call(
        _k1_body,
        grid=(_GRID,),
        in_specs=[
            pl.BlockSpec((_BN, _D), lambda i: (i, 0)),
            pl.BlockSpec((_D, _D), lambda i: (0, 0)),
            pl.BlockSpec((_D, 1), lambda i: (0, 0)),
            pl.BlockSpec((_D, 1), lambda i: (0, 0)),
        ],
        out_specs=[
            pl.BlockSpec((2, _BN, _H), lambda i: (0, i, 0)),
            pl.BlockSpec((_BN, 1), lambda i: (i, 0)),
            pl.BlockSpec((_BN, 1), lambda i: (i, 0)),
            pl.BlockSpec((1, 2), lambda i: (0, 0)),
        ],
        out_shape=[
            jax.ShapeDtypeStruct((2, _NP, _H), jnp.float32),
            jax.ShapeDtypeStruct((_NP, 1), jnp.float32),
            jax.ShapeDtypeStruct((_NP, 1), jnp.float32),
            jax.ShapeDtypeStruct((1, 2), jnp.float32),
        ],
    )(x, w, a_s, a_d)


def _run_k2(acc, den, b, w, a_s, a_d):
    return pl.pallas_call(
        _k2_body,
        grid=(_GRID,),
        in_specs=[
            pl.BlockSpec((2, _BN, _H), lambda i: (0, i, 0)),
            pl.BlockSpec((_BN, 1), lambda i: (i, 0)),
            pl.BlockSpec((1, _D), lambda i: (0, 0)),
            pl.BlockSpec((_D, _D), lambda i: (0, 0)),
            pl.BlockSpec((_D, 1), lambda i: (0, 0)),
            pl.BlockSpec((_D, 1), lambda i: (0, 0)),
        ],
        out_specs=[
            pl.BlockSpec((2, _BN, _H), lambda i: (0, i, 0)),
            pl.BlockSpec((_BN, 1), lambda i: (i, 0)),
            pl.BlockSpec((_BN, 1), lambda i: (i, 0)),
            pl.BlockSpec((1, 2), lambda i: (0, 0)),
        ],
        out_shape=[
            jax.ShapeDtypeStruct((2, _NP, _H), jnp.float32),
            jax.ShapeDtypeStruct((_NP, 1), jnp.float32),
            jax.ShapeDtypeStruct((_NP, 1), jnp.float32),
            jax.ShapeDtypeStruct((1, 2), jnp.float32),
        ],
    )(acc, den, b, w, a_s, a_d)


def _run_kemb(acc, den, b):
    return pl.pallas_call(
        _kemb_body,
        grid=(_GRID,),
        in_specs=[
            pl.BlockSpec((2, _BN, _H), lambda i: (0, i, 0)),
            pl.BlockSpec((_BN, 1), lambda i: (i, 0)),
            pl.BlockSpec((1, _D), lambda i: (0, 0)),
        ],
        out_specs=[
            pl.BlockSpec((_BN, _D), lambda i: (i, 0)),
            pl.BlockSpec((2, _BN, _H), lambda i: (0, i, 0)),
        ],
        out_shape=[
            jax.ShapeDtypeStruct((_NP, _D), jnp.float32),
            jax.ShapeDtypeStruct((2, _NP, _H), jnp.float32),
        ],
    )(acc, den, b)


def _run_k3(accr, denr, embt, acc2a, den2a, b2, disc_w, disc_b):
    return pl.pallas_call(
        _k3_body,
        grid=(_GRID,),
        in_specs=[
            pl.BlockSpec((2, _BN, _H), lambda i: (0, i, 0)),
            pl.BlockSpec((_BN, 1), lambda i: (i, 0)),
            pl.BlockSpec((2, _BN, _H), lambda i: (0, i, 0)),
            pl.BlockSpec((2, _BN, _H), lambda i: (0, i, 0)),
            pl.BlockSpec((_BN, 1), lambda i: (i, 0)),
            pl.BlockSpec((1, _D), lambda i: (0, 0)),
            pl.BlockSpec((_D, _D), lambda i: (0, 0)),
            pl.BlockSpec((1, 1), lambda i: (0, 0)),
        ],
        out_specs=pl.BlockSpec((_BN, 2), lambda i: (i, 0)),
        out_shape=jax.ShapeDtypeStruct((_NP, 2), jnp.float32),
    )(accr, denr, embt, acc2a, den2a, b2, disc_w, disc_b)


@functools.lru_cache(maxsize=None)
def _make_gat_pass(m_rows, e_true):
    rpw = m_rows // 16
    assert rpw % 2 == 0
    mesh = plsc.VectorSubcoreMesh(core_axis_name="c", subcore_axis_name="s")

    @functools.partial(
        pl.kernel,
        out_type=(
            jax.ShapeDtypeStruct((2, _NP, _H), jnp.float32),
            jax.ShapeDtypeStruct((_NP,), jnp.float32),
        ),
        mesh=mesh,
        compiler_params=pltpu.CompilerParams(needs_layout_passes=False),
        scratch_types=[
            pltpu.VMEM((16,), jnp.float32),
            pltpu.VMEM((128,), jnp.int32),
            pltpu.VMEM((128,), jnp.int32),
            pltpu.VMEM((128,), jnp.int32),
            pltpu.VMEM((128,), jnp.int32),
            pltpu.VMEM((128,), jnp.int32),
            pltpu.VMEM((128,), jnp.int32),
            pltpu.VMEM((128,), jnp.int32),
            pltpu.VMEM((128,), jnp.int32),
            pltpu.VMEM((128,), jnp.float32),
            pltpu.VMEM((128,), jnp.float32),
            pltpu.VMEM((128,), jnp.float32),
            pltpu.VMEM((128,), jnp.float32),
            pltpu.VMEM((128,), jnp.float32),
            pltpu.VMEM((128,), jnp.float32),
            pltpu.VMEM((128, _H), jnp.float32),
            pltpu.VMEM((128, _H), jnp.float32),
            pltpu.VMEM((_TPB,), jnp.float32),
            pltpu.SemaphoreType.DMA,
            pltpu.SemaphoreType.DMA,
            pltpu.VMEM_SHARED((_NP, _H), jnp.float32),
            pltpu.VMEM_SHARED((_NP,), jnp.float32),
        ],
    )
    def gat_pass(as_hbm, ad_hbm, tab_hbm, sg_hbm, dg_hbm, ds_hbm, m_hbm,
                 acc_hbm, den_hbm,
                 m_v, sgb0, sgb1, sgo0, sgo1, dgb0, dgb1, dsb0, dsb1,
                 asg0, asg1, adg0, adg1, exb0, exb1, rows0, rows1, zb,
                 isem, gsem, acc_sp, den_sp):
        c = lax.axis_index("c")
        s = lax.axis_index("s")
        z16 = jnp.zeros((16,), jnp.float32)
        set0 = (sgb0, sgo0, dgb0, dsb0, asg0, adg0, exb0, rows0)
        set1 = (sgb1, sgo1, dgb1, dsb1, asg1, adg1, exb1, rows1)

        pltpu.sync_copy(m_hbm, m_v)
        e0 = s * rpw * 128
        coff = c * _NP

        def _zrow(i, carry):
            for k in range(_H // 16):
                rows0[i, pl.ds(k * 16, 16)] = z16
            return carry

        lax.fori_loop(0, 128, _zrow, 0)

        def _zcp(t, carry):
            pltpu.sync_copy(rows0, acc_sp.at[pl.ds(s * _TPB + t * 128, 128)])
            return carry

        lax.fori_loop(0, _TPB // 128, _zcp, 0)

        def _zden(k, carry):
            zb[pl.ds(k * 16, 16)] = z16
            return carry

        lax.fori_loop(0, _TPB // 16, _zden, 0)
        pltpu.sync_copy(zb, den_sp.at[pl.ds(s * _TPB, _TPB)])
        plsc.subcore_barrier()

        mv = m_v[...]

        def _fire_idx(j, st):
            sgb, sgo, dgb, dsb = st[0], st[1], st[2], st[3]
            o = e0 + j * 128
            pltpu.async_copy(sg_hbm.at[pl.ds(o, 128)], sgb, isem)
            pltpu.async_copy(dg_hbm.at[pl.ds(o, 128)], dgb, isem)
            pltpu.async_copy(ds_hbm.at[pl.ds(o, 128)], dsb, isem)

        def _wait_idx(st):
            for buf in (st[0], st[2], st[3]):
                pltpu.make_async_copy(sg_hbm.at[pl.ds(0, 128)], buf,
                                      isem).wait()

        def _fire_gather(st):
            sgb, sgo, dgb = st[0], st[1], st[2]
            asg, adg, rows = st[4], st[5], st[7]
            for k in range(8):
                sgo[pl.ds(k * 16, 16)] = sgb[pl.ds(k * 16, 16)] + coff
            pltpu.async_copy(tab_hbm.at[sgo], rows, gsem)
            pltpu.async_copy(as_hbm.at[sgb], asg, gsem)
            pltpu.async_copy(ad_hbm.at[dgb], adg, gsem)

        def _wait_gather(st):
            asg, adg, rows = st[4], st[5], st[7]
            pltpu.make_async_copy(tab_hbm.at[pl.ds(0, 128)], rows,
                                  gsem).wait()
            pltpu.make_async_copy(as_hbm.at[pl.ds(0, 128)], asg, gsem).wait()
            pltpu.make_async_copy(as_hbm.at[pl.ds(0, 128)], adg, gsem).wait()

        def _process(j, st):
            dsb, asg, adg, exb, rows = st[3], st[4], st[5], st[6], st[7]
            base = e0 + j * 128
            for k in range(8):
                z = asg[pl.ds(k * 16, 16)] + adg[pl.ds(k * 16, 16)]
                e = jnp.where(z >= 0, z, 0.2 * z)
                ex = jnp.exp(e - mv)
                eidx = base + k * 16 + lax.iota(jnp.int32, 16)
                ex = jnp.where(eidx < e_true, ex, 0.0)
                exb[pl.ds(k * 16, 16)] = ex

            @pl.when(c == 0)
            def _():
                pltpu.sync_copy(exb, den_sp.at[dsb], add=True)

            def _scale(i, icarry):
                wv = plsc.load_gather(exb, [jnp.zeros((16,), jnp.int32) + i])
                for k in range(_H // 16):
                    rows[i, pl.ds(k * 16, 16)] = (
                        rows[i, pl.ds(k * 16, 16)] * wv)
                return icarry

            lax.fori_loop(0, 128, _scale, 0)
            pltpu.sync_copy(rows, acc_sp.at[dsb], add=True)

        def _half(b, cur, nxt):
            @pl.when(b + 1 < rpw)
            def _():
                _wait_idx(nxt)
                _fire_gather(nxt)

            _wait_gather(cur)
            _process(b, cur)

            @pl.when(b + 2 < rpw)
            def _():
                _fire_idx(b + 2, cur)

        _fire_idx(0, set0)
        _wait_idx(set0)
        _fire_gather(set0)
        _fire_idx(1, set1)

        def _outer(j2, carry):
            _half(2 * j2, set0, set1)
            _half(2 * j2 + 1, set1, set0)
            return carry

        lax.fori_loop(0, rpw // 2, _outer, 0)
        plsc.subcore_barrier()

        def _ocp(t, carry):
            rr = s * _TPB + t * 128
            pltpu.sync_copy(acc_sp.at[pl.ds(rr, 128)],
                            acc_hbm.at[c, pl.ds(rr, 128)])
            return carry

        lax.fori_loop(0, _TPB // 128, _ocp, 0)

        @pl.when(c == 0)
        def _():
            pltpu.sync_copy(den_sp.at[pl.ds(s * _TPB, _TPB)],
                            den_hbm.at[pl.ds(s * _TPB, _TPB)])

    return gat_pass


def kernel(x, edge_index, spatial_list, wnn_list, W1, a_src1, a_dst1, b1,
           W2, a_src2, a_dst2, b2, disc_W, disc_b):
    i32 = jnp.int32
    f32 = jnp.float32
    n = x.shape[0]

    loops = jnp.arange(n, dtype=i32)
    perm = jax.random.permutation(jax.random.key(42), n).astype(i32)

    e1_true = edge_index.shape[1] + n
    ep1 = ((e1_true + 4095) // 4096) * 4096
    er_true = edge_index.shape[1]
    epr = ((er_true + 4095) // 4096) * 4096

    def pad1d(a, ep):
        a = a.astype(i32)
        return jnp.concatenate([a, jnp.zeros((ep - a.shape[0],), i32)])

    src_w = jnp.concatenate([wnn_list[0].astype(i32), loops])
    dst_w = jnp.concatenate([wnn_list[1].astype(i32), loops])
    src_s = jnp.concatenate([spatial_list[0].astype(i32), loops])
    dst_s = jnp.concatenate([spatial_list[1].astype(i32), loops])

    sgw1 = pad1d(src_w, ep1)
    dgw1 = pad1d(dst_w, ep1)
    sgw2 = pad1d(perm[src_w], ep1)
    dgw2 = pad1d(perm[dst_w], ep1)
    sgs = pad1d(src_s, ep1)
    dgs = pad1d(dst_s, ep1)
    sgr = pad1d(edge_index[1], epr)
    dsr = pad1d(edge_index[0], epr)

    xp = jnp.pad(x.astype(f32), ((0, _NP - n), (0, 0)))
    as1_2d = a_src1.reshape(_D, 1).astype(f32)
    ad1_2d = a_dst1.reshape(_D, 1).astype(f32)
    as2_2d = a_src2.reshape(_D, 1).astype(f32)
    ad2_2d = a_dst2.reshape(_D, 1).astype(f32)
    b1_2d = b1.reshape(1, _D).astype(f32)
    b2_2d = b2.reshape(1, _D).astype(f32)
    db_2d = disc_b.reshape(1, 1).astype(f32)

    def mvec(m):
        return jnp.full((16,), jnp.maximum(m[0, 0] + m[0, 1], 0.0), f32)

    gat_pass = _make_gat_pass(ep1 // 128, e1_true)
    readout_pass = _make_gat_pass(epr // 128, er_true)

    h1t, as1, ad1, m1 = _run_k1(xp, W1.astype(f32), as1_2d, ad1_2d)
    tab1 = h1t.reshape(2 * _NP, _H)
    as1f = as1.reshape(_NP)
    ad1f = ad1.reshape(_NP)
    m1v = mvec(m1)

    acc11, den11 = gat_pass(as1f, ad1f, tab1, sgw1, dgw1, dgw1, m1v)
    acc12, den12 = gat_pass(as1f, ad1f, tab1, sgw2, dgw2, dgw1, m1v)

    h2t1, as21, ad21, m21 = _run_k2(acc11, den11.reshape(_NP, 1), b1_2d,
                                    W2.astype(f32), as2_2d, ad2_2d)
    h2t2, as22, ad22, m22 = _run_k2(acc12, den12.reshape(_NP, 1), b1_2d,
                                    W2.astype(f32), as2_2d, ad2_2d)

    acc21, den21 = gat_pass(as21.reshape(_NP), ad21.reshape(_NP),
                            h2t1.reshape(2 * _NP, _H), sgs, dgs, dgs,
                            mvec(m21))
    acc22, den22 = gat_pass(as22.reshape(_NP), ad22.reshape(_NP),
                            h2t2.reshape(2 * _NP, _H), sgs, dgs, dgs,
                            mvec(m22))

    h2_full, embt = _run_kemb(acc21, den21.reshape(_NP, 1), b2_2d)

    zn = jnp.zeros((_NP,), f32)
    z16 = jnp.zeros((16,), f32)
    accr, denr = readout_pass(zn, zn, embt.reshape(2 * _NP, _H),
                              sgr, sgr, dsr, z16)

    ret_full = _run_k3(accr, denr.reshape(_NP, 1), embt, acc22,
                       den22.reshape(_NP, 1), b2_2d,
                       disc_W.astype(f32), db_2d)

    return h2_full[:n], ret_full[:n]


# speedup vs baseline: 6.2758x; 1.0487x over previous
import functools

import jax
import jax.numpy as jnp
from jax import lax
from jax.experimental import pallas as pl
from jax.experimental.pallas import tpu as pltpu
from jax.experimental.pallas import tpu_sc as plsc

_N = 10000
_NP = 10240
_D = 256
_H = 128
_BN = 1280
_GRID = _NP // _BN
_TPB = 640


def _mm_alphas_body(x, w_ref, as_ref, ad_ref, h_ref, s_ref, d_ref, m_ref):
    h = jnp.dot(x, w_ref[...], preferred_element_type=jnp.float32)
    h_ref[...] = jnp.stack([h[:, :_H], h[:, _H:]])
    s = jnp.dot(h, as_ref[...], preferred_element_type=jnp.float32)
    d = jnp.dot(h, ad_ref[...], preferred_element_type=jnp.float32)
    s_ref[...] = s
    d_ref[...] = d
    cur = jnp.concatenate([jnp.max(s).reshape(1, 1), jnp.max(d).reshape(1, 1)],
                          axis=1)
    i = pl.program_id(0)

    @pl.when(i == 0)
    def _():
        m_ref[...] = cur

    @pl.when(i > 0)
    def _():
        m_ref[...] = jnp.maximum(m_ref[...], cur)


def _k1_body(x_ref, w_ref, as_ref, ad_ref, h_ref, s_ref, d_ref, m_ref):
    _mm_alphas_body(x_ref[...], w_ref, as_ref, ad_ref, h_ref, s_ref, d_ref,
                    m_ref)


def _k2_body(a_ref, den_ref, b_ref, w_ref, as_ref, ad_ref,
             h_ref, s_ref, d_ref, m_ref):
    acc = a_ref[...]
    xcat = jnp.concatenate([acc[0], acc[1]], axis=1)
    den = den_ref[...]
    dsum = den[:, 0:1] + den[:, 1:2]
    xin = jnp.where(dsum > 0, xcat / dsum, 0.0) + b_ref[...]
    xin = jnp.maximum(xin, 0.0)
    _mm_alphas_body(xin, w_ref, as_ref, ad_ref, h_ref, s_ref, d_ref, m_ref)


def _kemb_body(a_ref, den_ref, b_ref, h2_ref, e_ref):
    acc = a_ref[...]
    xcat = jnp.concatenate([acc[0], acc[1]], axis=1)
    den = den_ref[...]
    h2 = xcat / (den[:, 0:1] + den[:, 1:2]) + b_ref[...]
    h2_ref[...] = h2
    emb = jnp.maximum(h2, 0.0)
    e_ref[...] = jnp.stack([emb[:, :_H], emb[:, _H:]])


def _k3_body(ar_ref, dr_ref, e_ref, a2_ref, d2_ref, b_ref, w_ref, db_ref,
             out_ref):
    ar = ar_ref[...]
    gpre = jnp.concatenate([ar[0], ar[1]], axis=1)
    dr = dr_ref[...]
    gpre = gpre / (dr[:, 0:1] + dr[:, 1:2])
    nrm = jnp.sqrt(jnp.sum(gpre * gpre, axis=1, keepdims=True))
    nrm = jnp.maximum(nrm, 1e-12)
    g = jax.nn.sigmoid(gpre / nrm)

    e = e_ref[...]
    emb = jnp.concatenate([e[0], e[1]], axis=1)
    a2 = a2_ref[...]
    d2 = d2_ref[...]
    emba = jnp.maximum(
        jnp.concatenate([a2[0], a2[1]], axis=1) / (d2[:, 0:1] + d2[:, 1:2])
        + b_ref[...],
        0.0)

    w = w_ref[...]
    db = db_ref[...]
    e1 = jnp.dot(emb, w, preferred_element_type=jnp.float32)
    sc1 = jnp.sum(e1 * g, axis=1, keepdims=True) + db
    e2 = jnp.dot(emba, w, preferred_element_type=jnp.float32)
    sc2 = jnp.sum(e2 * g, axis=1, keepdims=True) + db
    out_ref[...] = jnp.concatenate([sc1, sc2], axis=1)


def _run_k1(x, w, a_s, a_d):
    return pl.pallas_call(
        _k1_body,
        grid=(_GRID,),
        in_specs=[
            pl.BlockSpec((_BN, _D), lambda i: (i, 0)),
            pl.BlockSpec((_D, _D), lambda i: (0, 0)),
            pl.BlockSpec((_D, 1), lambda i: (0, 0)),
            pl.BlockSpec((_D, 1), lambda i: (0, 0)),
        ],
        out_specs=[
            pl.BlockSpec((2, _BN, _H), lambda i: (0, i, 0)),
            pl.BlockSpec((_BN, 1), lambda i: (i, 0)),
            pl.BlockSpec((_BN, 1), lambda i: (i, 0)),
            pl.BlockSpec((1, 2), lambda i: (0, 0)),
        ],
        out_shape=[
            jax.ShapeDtypeStruct((2, _NP, _H), jnp.float32),
            jax.ShapeDtypeStruct((_NP, 1), jnp.float32),
            jax.ShapeDtypeStruct((_NP, 1), jnp.float32),
            jax.ShapeDtypeStruct((1, 2), jnp.float32),
        ],
    )(x, w, a_s, a_d)


def _run_k2(acc, den, b, w, a_s, a_d):
    return pl.pallas_call(
        _k2_body,
        grid=(_GRID,),
        in_specs=[
            pl.BlockSpec((2, _BN, _H), lambda i: (0, i, 0)),
            pl.BlockSpec((_BN, 2), lambda i: (i, 0)),
            pl.BlockSpec((1, _D), lambda i: (0, 0)),
            pl.BlockSpec((_D, _D), lambda i: (0, 0)),
            pl.BlockSpec((_D, 1), lambda i: (0, 0)),
            pl.BlockSpec((_D, 1), lambda i: (0, 0)),
        ],
        out_specs=[
            pl.BlockSpec((2, _BN, _H), lambda i: (0, i, 0)),
            pl.BlockSpec((_BN, 1), lambda i: (i, 0)),
            pl.BlockSpec((_BN, 1), lambda i: (i, 0)),
            pl.BlockSpec((1, 2), lambda i: (0, 0)),
        ],
        out_shape=[
            jax.ShapeDtypeStruct((2, _NP, _H), jnp.float32),
            jax.ShapeDtypeStruct((_NP, 1), jnp.float32),
            jax.ShapeDtypeStruct((_NP, 1), jnp.float32),
            jax.ShapeDtypeStruct((1, 2), jnp.float32),
        ],
    )(acc, den, b, w, a_s, a_d)


def _run_kemb(acc, den, b):
    return pl.pallas_call(
        _kemb_body,
        grid=(_GRID,),
        in_specs=[
            pl.BlockSpec((2, _BN, _H), lambda i: (0, i, 0)),
            pl.BlockSpec((_BN, 2), lambda i: (i, 0)),
            pl.BlockSpec((1, _D), lambda i: (0, 0)),
        ],
        out_specs=[
            pl.BlockSpec((_BN, _D), lambda i: (i, 0)),
            pl.BlockSpec((2, _BN, _H), lambda i: (0, i, 0)),
        ],
        out_shape=[
            jax.ShapeDtypeStruct((_NP, _D), jnp.float32),
            jax.ShapeDtypeStruct((2, _NP, _H), jnp.float32),
        ],
    )(acc, den, b)


def _run_k3(accr, denr, embt, acc2a, den2a, b2, disc_w, disc_b):
    return pl.pallas_call(
        _k3_body,
        grid=(_GRID,),
        in_specs=[
            pl.BlockSpec((2, _BN, _H), lambda i: (0, i, 0)),
            pl.BlockSpec((_BN, 2), lambda i: (i, 0)),
            pl.BlockSpec((2, _BN, _H), lambda i: (0, i, 0)),
            pl.BlockSpec((2, _BN, _H), lambda i: (0, i, 0)),
            pl.BlockSpec((_BN, 2), lambda i: (i, 0)),
            pl.BlockSpec((1, _D), lambda i: (0, 0)),
            pl.BlockSpec((_D, _D), lambda i: (0, 0)),
            pl.BlockSpec((1, 1), lambda i: (0, 0)),
        ],
        out_specs=pl.BlockSpec((_BN, 2), lambda i: (i, 0)),
        out_shape=jax.ShapeDtypeStruct((_NP, 2), jnp.float32),
    )(accr, denr, embt, acc2a, den2a, b2, disc_w, disc_b)


@functools.lru_cache(maxsize=None)
def _make_gat_pass(m_rows, e_true):
    rpw = m_rows // 16
    assert rpw % 2 == 0
    mesh = plsc.VectorSubcoreMesh(core_axis_name="c", subcore_axis_name="s")

    @functools.partial(
        pl.kernel,
        out_type=(
            jax.ShapeDtypeStruct((2, _NP, _H), jnp.float32),
            jax.ShapeDtypeStruct((2, _NP), jnp.float32),
        ),
        mesh=mesh,
        compiler_params=pltpu.CompilerParams(needs_layout_passes=False),
        scratch_types=[
            pltpu.VMEM((16,), jnp.float32),
            pltpu.VMEM((128,), jnp.int32),
            pltpu.VMEM((128,), jnp.int32),
            pltpu.VMEM((128,), jnp.int32),
            pltpu.VMEM((128,), jnp.int32),
            pltpu.VMEM((128,), jnp.int32),
            pltpu.VMEM((128,), jnp.int32),
            pltpu.VMEM((128,), jnp.int32),
            pltpu.VMEM((128,), jnp.int32),
            pltpu.VMEM((128,), jnp.int32),
            pltpu.VMEM((128,), jnp.int32),
            pltpu.VMEM((128,), jnp.float32),
            pltpu.VMEM((128,), jnp.float32),
            pltpu.VMEM((128,), jnp.float32),
            pltpu.VMEM((128,), jnp.float32),
            pltpu.VMEM((128,), jnp.float32),
            pltpu.VMEM((128,), jnp.float32),
            pltpu.VMEM((128, _H), jnp.float32),
            pltpu.VMEM((128, _H), jnp.float32),
            pltpu.VMEM((_TPB,), jnp.float32),
            pltpu.SemaphoreType.DMA,
            pltpu.SemaphoreType.DMA,
            pltpu.SemaphoreType.DMA,
            pltpu.SemaphoreType.DMA,
            pltpu.SemaphoreType.DMA,
            pltpu.VMEM_SHARED((_NP, _H), jnp.float32),
            pltpu.VMEM_SHARED((_NP,), jnp.float32),
        ],
    )
    def gat_pass(as_hbm, ad_hbm, tab_hbm, sg_hbm, dg_hbm, ds_hbm, m_hbm,
                 acc_hbm, den_hbm,
                 m_v, sgb0, sgb1, sgo0, sgo1, dgb0, dgb1, dsb0, dsb1,
                 dsc0, dsc1, asg0, asg1, adg0, adg1, exb0, exb1,
                 rows0, rows1, zb,
                 isem, asem, gsem, ssem, dsem, acc_sp, den_sp):
        c = lax.axis_index("c")
        s = lax.axis_index("s")
        z16 = jnp.zeros((16,), jnp.float32)
        set0 = (sgb0, sgo0, dgb0, dsb0, asg0, adg0, exb0, rows0, dsc0)
        set1 = (sgb1, sgo1, dgb1, dsb1, asg1, adg1, exb1, rows1, dsc1)

        pltpu.sync_copy(m_hbm, m_v)
        e0 = s * rpw * 128
        coff = c * _NP

        def _zrow(i, carry):
            for k in range(_H // 16):
                rows0[i, pl.ds(k * 16, 16)] = z16
            return carry

        lax.fori_loop(0, 128, _zrow, 0)

        def _zcp(t, carry):
            pltpu.sync_copy(rows0, acc_sp.at[pl.ds(s * _TPB + t * 128, 128)])
            return carry

        lax.fori_loop(0, _TPB // 128, _zcp, 0)

        def _zden(k, carry):
            zb[pl.ds(k * 16, 16)] = z16
            return carry

        lax.fori_loop(0, _TPB // 16, _zden, 0)
        pltpu.sync_copy(zb, den_sp.at[pl.ds(s * _TPB, _TPB)])
        plsc.subcore_barrier()

        mv = m_v[...]

        def _fire_idx(j, st):
            sgb, dgb, dsb = st[0], st[2], st[3]
            o = e0 + j * 128
            pltpu.async_copy(sg_hbm.at[pl.ds(o, 128)], sgb, isem)
            pltpu.async_copy(dg_hbm.at[pl.ds(o, 128)], dgb, isem)
            pltpu.async_copy(ds_hbm.at[pl.ds(o, 128)], dsb, isem)

        def _wait_idx(st):
            for buf in (st[0], st[2], st[3]):
                pltpu.make_async_copy(sg_hbm.at[pl.ds(0, 128)], buf,
                                      isem).wait()

        def _fire_gather(st):
            sgb, sgo, dgb = st[0], st[1], st[2]
            asg, adg, rows = st[4], st[5], st[7]
            for k in range(8):
                sgo[pl.ds(k * 16, 16)] = sgb[pl.ds(k * 16, 16)] + coff
            pltpu.async_copy(tab_hbm.at[sgo], rows, gsem)
            pltpu.async_copy(as_hbm.at[sgb], asg, asem)
            pltpu.async_copy(ad_hbm.at[dgb], adg, asem)

        def _wait_gather(st):
            asg, adg, rows = st[4], st[5], st[7]
            pltpu.make_async_copy(tab_hbm.at[pl.ds(0, 128)], rows,
                                  gsem).wait()
            pltpu.make_async_copy(as_hbm.at[pl.ds(0, 128)], asg, asem).wait()
            pltpu.make_async_copy(as_hbm.at[pl.ds(0, 128)], adg, asem).wait()

        def _process(j, st, par):
            dsb, asg, adg, exb, rows, dsc = (st[3], st[4], st[5], st[6],
                                             st[7], st[8])
            base = e0 + j * 128
            for k in range(8):
                dsc[pl.ds(k * 16, 16)] = dsb[pl.ds(k * 16, 16)]
                z = asg[pl.ds(k * 16, 16)] + adg[pl.ds(k * 16, 16)]
                e = jnp.where(z >= 0, z, 0.2 * z)
                ex = jnp.exp(e - mv)
                eidx = base + k * 16 + lax.iota(jnp.int32, 16)
                ex = jnp.where(eidx < e_true, ex, 0.0)
                exb[pl.ds(k * 16, 16)] = ex

            @pl.when(c == par)
            def _():
                pltpu.async_copy(exb, den_sp.at[dsc], dsem, add=True)

            def _scale(i2, icarry):
                i = 2 * i2
                wv0 = plsc.load_gather(exb, [jnp.zeros((16,), jnp.int32) + i])
                wv1 = plsc.load_gather(exb,
                                       [jnp.zeros((16,), jnp.int32) + i + 1])
                for k in range(_H // 16):
                    rows[i, pl.ds(k * 16, 16)] = (
                        rows[i, pl.ds(k * 16, 16)] * wv0)
                for k in range(_H // 16):
                    rows[i + 1, pl.ds(k * 16, 16)] = (
                        rows[i + 1, pl.ds(k * 16, 16)] * wv1)
                return icarry

            lax.fori_loop(0, 64, _scale, 0)
            pltpu.async_copy(rows, acc_sp.at[dsc], ssem, add=True)

        def _wait_scatter(st, par):
            rows, exb = st[7], st[6]
            pltpu.make_async_copy(tab_hbm.at[pl.ds(0, 128)], rows,
                                  ssem).wait()

            @pl.when(c == par)
            def _():
                pltpu.make_async_copy(as_hbm.at[pl.ds(0, 128)], exb,
                                      dsem).wait()

        def _half(b, cur, nxt, pc, pn):
            @pl.when(b + 1 < rpw)
            def _():
                @pl.when(b >= 1)
                def _():
                    _wait_scatter(nxt, pn)

                _wait_idx(nxt)
                _fire_gather(nxt)

            _wait_gather(cur)
            _process(b, cur, pc)

            @pl.when(b + 2 < rpw)
            def _():
                _fire_idx(b + 2, cur)

        _fire_idx(0, set0)
        _wait_idx(set0)
        _fire_gather(set0)
        _fire_idx(1, set1)

        def _outer(j2, carry):
            _half(2 * j2, set0, set1, 0, 1)
            _half(2 * j2 + 1, set1, set0, 1, 0)
            return carry

        lax.fori_loop(0, rpw // 2, _outer, 0)
        _wait_scatter(set0, 0)
        _wait_scatter(set1, 1)
        plsc.subcore_barrier()

        def _ocp(t, carry):
            rr = s * _TPB + t * 128
            pltpu.sync_copy(acc_sp.at[pl.ds(rr, 128)],
                            acc_hbm.at[c, pl.ds(rr, 128)])
            return carry

        lax.fori_loop(0, _TPB // 128, _ocp, 0)

        pltpu.sync_copy(den_sp.at[pl.ds(s * _TPB, _TPB)],
                        den_hbm.at[c, pl.ds(s * _TPB, _TPB)])

    return gat_pass


def kernel(x, edge_index, spatial_list, wnn_list, W1, a_src1, a_dst1, b1,
           W2, a_src2, a_dst2, b2, disc_W, disc_b):
    i32 = jnp.int32
    f32 = jnp.float32
    n = x.shape[0]

    loops = jnp.arange(n, dtype=i32)
    perm = jax.random.permutation(jax.random.key(42), n).astype(i32)

    e1_true = edge_index.shape[1] + n
    ep1 = ((e1_true + 4095) // 4096) * 4096
    er_true = edge_index.shape[1]
    epr = ((er_true + 4095) // 4096) * 4096

    def pad1d(a, ep):
        a = a.astype(i32)
        return jnp.concatenate([a, jnp.zeros((ep - a.shape[0],), i32)])

    src_w = jnp.concatenate([wnn_list[0].astype(i32), loops])
    dst_w = jnp.concatenate([wnn_list[1].astype(i32), loops])
    src_s = jnp.concatenate([spatial_list[0].astype(i32), loops])
    dst_s = jnp.concatenate([spatial_list[1].astype(i32), loops])

    sgw1 = pad1d(src_w, ep1)
    dgw1 = pad1d(dst_w, ep1)
    sgw2 = pad1d(perm[src_w], ep1)
    dgw2 = pad1d(perm[dst_w], ep1)
    sgs = pad1d(src_s, ep1)
    dgs = pad1d(dst_s, ep1)
    sgr = pad1d(edge_index[1], epr)
    dsr = pad1d(edge_index[0], epr)

    xp = jnp.pad(x.astype(f32), ((0, _NP - n), (0, 0)))
    as1_2d = a_src1.reshape(_D, 1).astype(f32)
    ad1_2d = a_dst1.reshape(_D, 1).astype(f32)
    as2_2d = a_src2.reshape(_D, 1).astype(f32)
    ad2_2d = a_dst2.reshape(_D, 1).astype(f32)
    b1_2d = b1.reshape(1, _D).astype(f32)
    b2_2d = b2.reshape(1, _D).astype(f32)
    db_2d = disc_b.reshape(1, 1).astype(f32)

    def mvec(m):
        return jnp.full((16,), jnp.maximum(m[0, 0] + m[0, 1], 0.0), f32)

    gat_pass = _make_gat_pass(ep1 // 128, e1_true)
    readout_pass = _make_gat_pass(epr // 128, er_true)

    h1t, as1, ad1, m1 = _run_k1(xp, W1.astype(f32), as1_2d, ad1_2d)
    tab1 = h1t.reshape(2 * _NP, _H)
    as1f = as1.reshape(_NP)
    ad1f = ad1.reshape(_NP)
    m1v = mvec(m1)

    acc11, den11 = gat_pass(as1f, ad1f, tab1, sgw1, dgw1, dgw1, m1v)
    acc12, den12 = gat_pass(as1f, ad1f, tab1, sgw2, dgw2, dgw1, m1v)

    h2t1, as21, ad21, m21 = _run_k2(acc11, den11.T, b1_2d,
                                    W2.astype(f32), as2_2d, ad2_2d)
    h2t2, as22, ad22, m22 = _run_k2(acc12, den12.T, b1_2d,
                                    W2.astype(f32), as2_2d, ad2_2d)

    acc21, den21 = gat_pass(as21.reshape(_NP), ad21.reshape(_NP),
                            h2t1.reshape(2 * _NP, _H), sgs, dgs, dgs,
                            mvec(m21))
    acc22, den22 = gat_pass(as22.reshape(_NP), ad22.reshape(_NP),
                            h2t2.reshape(2 * _NP, _H), sgs, dgs, dgs,
                            mvec(m22))

    h2_full, embt = _run_kemb(acc21, den21.T, b2_2d)

    zn = jnp.zeros((_NP,), f32)
    z16 = jnp.zeros((16,), f32)
    accr, denr = readout_pass(zn, zn, embt.reshape(2 * _NP, _H),
                              sgr, sgr, dsr, z16)

    ret_full = _run_k3(accr, denr.T, embt, acc22,
                       den22.T, b2_2d,
                       disc_W.astype(f32), db_2d)

    return h2_full[:n], ret_full[:n]


# speedup vs baseline: 6.4326x; 1.0250x over previous
import functools

import jax
import jax.numpy as jnp
from jax import lax
from jax.experimental import pallas as pl
from jax.experimental.pallas import tpu as pltpu
from jax.experimental.pallas import tpu_sc as plsc

_N = 10000
_NP = 10240
_D = 256
_H = 128
_BN = 1280
_GRID = _NP // _BN
_TPB = 640
_EB = 112
_KG = _EB // 16


def _mm_alphas_body(x, w_ref, as_ref, ad_ref, h_ref, s_ref, d_ref, m_ref):
    h = jnp.dot(x, w_ref[...], preferred_element_type=jnp.float32)
    h_ref[...] = jnp.stack([h[:, :_H], h[:, _H:]])
    s = jnp.dot(h, as_ref[...], preferred_element_type=jnp.float32)
    d = jnp.dot(h, ad_ref[...], preferred_element_type=jnp.float32)
    s_ref[...] = s
    d_ref[...] = d
    cur = jnp.concatenate([jnp.max(s).reshape(1, 1), jnp.max(d).reshape(1, 1)],
                          axis=1)
    i = pl.program_id(0)

    @pl.when(i == 0)
    def _():
        m_ref[...] = cur

    @pl.when(i > 0)
    def _():
        m_ref[...] = jnp.maximum(m_ref[...], cur)


def _k1_body(x_ref, w_ref, as_ref, ad_ref, h_ref, s_ref, d_ref, m_ref):
    _mm_alphas_body(x_ref[...], w_ref, as_ref, ad_ref, h_ref, s_ref, d_ref,
                    m_ref)


def _k2_body(a_ref, den_ref, b_ref, w_ref, as_ref, ad_ref,
             h_ref, s_ref, d_ref, m_ref):
    acc = a_ref[...]
    xcat = jnp.concatenate([acc[0], acc[1]], axis=1)
    den = den_ref[...]
    dsum = den[:, 0:1] + den[:, 1:2]
    xin = jnp.where(dsum > 0, xcat / dsum, 0.0) + b_ref[...]
    xin = jnp.maximum(xin, 0.0)
    _mm_alphas_body(xin, w_ref, as_ref, ad_ref, h_ref, s_ref, d_ref, m_ref)


def _kemb_body(a_ref, den_ref, b_ref, h2_ref, e_ref):
    acc = a_ref[...]
    xcat = jnp.concatenate([acc[0], acc[1]], axis=1)
    den = den_ref[...]
    h2 = xcat / (den[:, 0:1] + den[:, 1:2]) + b_ref[...]
    h2_ref[...] = h2
    emb = jnp.maximum(h2, 0.0)
    e_ref[...] = jnp.stack([emb[:, :_H], emb[:, _H:]])


def _k3_body(ar_ref, dr_ref, e_ref, a2_ref, d2_ref, b_ref, w_ref, db_ref,
             out_ref):
    ar = ar_ref[...]
    gpre = jnp.concatenate([ar[0], ar[1]], axis=1)
    dr = dr_ref[...]
    gpre = gpre / (dr[:, 0:1] + dr[:, 1:2])
    nrm = jnp.sqrt(jnp.sum(gpre * gpre, axis=1, keepdims=True))
    nrm = jnp.maximum(nrm, 1e-12)
    g = jax.nn.sigmoid(gpre / nrm)

    e = e_ref[...]
    emb = jnp.concatenate([e[0], e[1]], axis=1)
    a2 = a2_ref[...]
    d2 = d2_ref[...]
    emba = jnp.maximum(
        jnp.concatenate([a2[0], a2[1]], axis=1) / (d2[:, 0:1] + d2[:, 1:2])
        + b_ref[...],
        0.0)

    w = w_ref[...]
    db = db_ref[...]
    e1 = jnp.dot(emb, w, preferred_element_type=jnp.float32)
    sc1 = jnp.sum(e1 * g, axis=1, keepdims=True) + db
    e2 = jnp.dot(emba, w, preferred_element_type=jnp.float32)
    sc2 = jnp.sum(e2 * g, axis=1, keepdims=True) + db
    out_ref[...] = jnp.concatenate([sc1, sc2], axis=1)


def _run_k1(x, w, a_s, a_d):
    return pl.pallas_call(
        _k1_body,
        grid=(_GRID,),
        in_specs=[
            pl.BlockSpec((_BN, _D), lambda i: (i, 0)),
            pl.BlockSpec((_D, _D), lambda i: (0, 0)),
            pl.BlockSpec((_D, 1), lambda i: (0, 0)),
            pl.BlockSpec((_D, 1), lambda i: (0, 0)),
        ],
        out_specs=[
            pl.BlockSpec((2, _BN, _H), lambda i: (0, i, 0)),
            pl.BlockSpec((_BN, 1), lambda i: (i, 0)),
            pl.BlockSpec((_BN, 1), lambda i: (i, 0)),
            pl.BlockSpec((1, 2), lambda i: (0, 0)),
        ],
        out_shape=[
            jax.ShapeDtypeStruct((2, _NP, _H), jnp.float32),
            jax.ShapeDtypeStruct((_NP, 1), jnp.float32),
            jax.ShapeDtypeStruct((_NP, 1), jnp.float32),
            jax.ShapeDtypeStruct((1, 2), jnp.float32),
        ],
    )(x, w, a_s, a_d)


def _run_k2(acc, den, b, w, a_s, a_d):
    return pl.pallas_call(
        _k2_body,
        grid=(_GRID,),
        in_specs=[
            pl.BlockSpec((2, _BN, _H), lambda i: (0, i, 0)),
            pl.BlockSpec((_BN, 2), lambda i: (i, 0)),
            pl.BlockSpec((1, _D), lambda i: (0, 0)),
            pl.BlockSpec((_D, _D), lambda i: (0, 0)),
            pl.BlockSpec((_D, 1), lambda i: (0, 0)),
            pl.BlockSpec((_D, 1), lambda i: (0, 0)),
        ],
        out_specs=[
            pl.BlockSpec((2, _BN, _H), lambda i: (0, i, 0)),
            pl.BlockSpec((_BN, 1), lambda i: (i, 0)),
            pl.BlockSpec((_BN, 1), lambda i: (i, 0)),
            pl.BlockSpec((1, 2), lambda i: (0, 0)),
        ],
        out_shape=[
            jax.ShapeDtypeStruct((2, _NP, _H), jnp.float32),
            jax.ShapeDtypeStruct((_NP, 1), jnp.float32),
            jax.ShapeDtypeStruct((_NP, 1), jnp.float32),
            jax.ShapeDtypeStruct((1, 2), jnp.float32),
        ],
    )(acc, den, b, w, a_s, a_d)


def _run_kemb(acc, den, b):
    return pl.pallas_call(
        _kemb_body,
        grid=(_GRID,),
        in_specs=[
            pl.BlockSpec((2, _BN, _H), lambda i: (0, i, 0)),
            pl.BlockSpec((_BN, 2), lambda i: (i, 0)),
            pl.BlockSpec((1, _D), lambda i: (0, 0)),
        ],
        out_specs=[
            pl.BlockSpec((_BN, _D), lambda i: (i, 0)),
            pl.BlockSpec((2, _BN, _H), lambda i: (0, i, 0)),
        ],
        out_shape=[
            jax.ShapeDtypeStruct((_NP, _D), jnp.float32),
            jax.ShapeDtypeStruct((2, _NP, _H), jnp.float32),
        ],
    )(acc, den, b)


def _run_k3(accr, denr, embt, acc2a, den2a, b2, disc_w, disc_b):
    return pl.pallas_call(
        _k3_body,
        grid=(_GRID,),
        in_specs=[
            pl.BlockSpec((2, _BN, _H), lambda i: (0, i, 0)),
            pl.BlockSpec((_BN, 2), lambda i: (i, 0)),
            pl.BlockSpec((2, _BN, _H), lambda i: (0, i, 0)),
            pl.BlockSpec((2, _BN, _H), lambda i: (0, i, 0)),
            pl.BlockSpec((_BN, 2), lambda i: (i, 0)),
            pl.BlockSpec((1, _D), lambda i: (0, 0)),
            pl.BlockSpec((_D, _D), lambda i: (0, 0)),
            pl.BlockSpec((1, 1), lambda i: (0, 0)),
        ],
        out_specs=pl.BlockSpec((_BN, 2), lambda i: (i, 0)),
        out_shape=jax.ShapeDtypeStruct((_NP, 2), jnp.float32),
    )(accr, denr, embt, acc2a, den2a, b2, disc_w, disc_b)


@functools.lru_cache(maxsize=None)
def _make_gat_pass(e_pad, e_true):
    rpw = e_pad // (16 * _EB)
    assert rpw % 6 == 0 and rpw >= 12
    mesh = plsc.VectorSubcoreMesh(core_axis_name="c", subcore_axis_name="s")

    @functools.partial(
        pl.kernel,
        out_type=(
            jax.ShapeDtypeStruct((2, _NP, _H), jnp.float32),
            jax.ShapeDtypeStruct((2, _NP), jnp.float32),
        ),
        mesh=mesh,
        compiler_params=pltpu.CompilerParams(needs_layout_passes=False),
        scratch_types=[
            pltpu.VMEM((16,), jnp.float32),
            pltpu.VMEM((_EB,), jnp.int32),
            pltpu.VMEM((_EB,), jnp.int32),
            pltpu.VMEM((_EB,), jnp.int32),
            pltpu.VMEM((_EB,), jnp.int32),
            pltpu.VMEM((_EB,), jnp.int32),
            pltpu.VMEM((_EB,), jnp.int32),
            pltpu.VMEM((_EB,), jnp.int32),
            pltpu.VMEM((_EB,), jnp.int32),
            pltpu.VMEM((_EB,), jnp.float32),
            pltpu.VMEM((_EB,), jnp.float32),
            pltpu.VMEM((_EB,), jnp.float32),
            pltpu.VMEM((_EB,), jnp.float32),
            pltpu.VMEM((_EB, _H), jnp.float32),
            pltpu.VMEM((_EB, _H), jnp.float32),
            pltpu.VMEM((_EB, _H), jnp.float32),
            pltpu.VMEM((_EB,), jnp.int32),
            pltpu.VMEM((_EB,), jnp.int32),
            pltpu.VMEM((_EB,), jnp.int32),
            pltpu.VMEM((_EB,), jnp.float32),
            pltpu.VMEM((_EB,), jnp.float32),
            pltpu.VMEM((_EB,), jnp.float32),
            pltpu.VMEM((_TPB,), jnp.float32),
            pltpu.SemaphoreType.DMA,
            pltpu.SemaphoreType.DMA,
            pltpu.SemaphoreType.DMA,
            pltpu.SemaphoreType.DMA,
            pltpu.SemaphoreType.DMA,
            pltpu.VMEM_SHARED((_NP, _H), jnp.float32),
            pltpu.VMEM_SHARED((_NP,), jnp.float32),
        ],
    )
    def gat_pass(as_hbm, ad_hbm, tab_hbm, sg_hbm, dg_hbm, ds_hbm, m_hbm,
                 acc_hbm, den_hbm,
                 m_v, sgb0, sgb1, sgo0, sgo1, dgb0, dgb1, dsb0, dsb1,
                 asg0, asg1, adg0, adg1,
                 rows0, rows1, rows2, dsc0, dsc1, dsc2, exb0, exb1, exb2,
                 zb, isem, asem, gsem, ssem, dsem, acc_sp, den_sp):
        c = lax.axis_index("c")
        s = lax.axis_index("s")
        z16 = jnp.zeros((16,), jnp.float32)
        sets = ((sgb0, sgo0, dgb0, dsb0, asg0, adg0),
                (sgb1, sgo1, dgb1, dsb1, asg1, adg1))
        rgs = ((rows0, dsc0, exb0), (rows1, dsc1, exb1), (rows2, dsc2, exb2))

        pltpu.sync_copy(m_hbm, m_v)
        e0 = s * rpw * _EB
        coff = c * _NP

        def _zrow(i, carry):
            for k in range(_H // 16):
                rows0[i, pl.ds(k * 16, 16)] = z16
            return carry

        lax.fori_loop(0, _EB, _zrow, 0)

        def _zcp(t, carry):
            pltpu.sync_copy(rows0.at[pl.ds(0, 80)],
                            acc_sp.at[pl.ds(s * _TPB + t * 80, 80)])
            return carry

        lax.fori_loop(0, _TPB // 80, _zcp, 0)

        def _zden(k, carry):
            zb[pl.ds(k * 16, 16)] = z16
            return carry

        lax.fori_loop(0, _TPB // 16, _zden, 0)
        pltpu.sync_copy(zb, den_sp.at[pl.ds(s * _TPB, _TPB)])
        plsc.subcore_barrier()

        mv = m_v[...]

        def _fire_idx(j, st):
            o = e0 + j * _EB
            pltpu.async_copy(sg_hbm.at[pl.ds(o, _EB)], st[0], isem)
            pltpu.async_copy(dg_hbm.at[pl.ds(o, _EB)], st[2], isem)
            pltpu.async_copy(ds_hbm.at[pl.ds(o, _EB)], st[3], isem)

        def _wait_idx(st):
            for buf in (st[0], st[2], st[3]):
                pltpu.make_async_copy(sg_hbm.at[pl.ds(0, _EB)], buf,
                                      isem).wait()

        def _fire_gather(st, rg):
            sgb, sgo, dgb, asg, adg = st[0], st[1], st[2], st[4], st[5]
            for k in range(_KG):
                sgo[pl.ds(k * 16, 16)] = sgb[pl.ds(k * 16, 16)] + coff
            pltpu.async_copy(tab_hbm.at[sgo], rg[0], gsem)
            pltpu.async_copy(as_hbm.at[sgb], asg, asem)
            pltpu.async_copy(ad_hbm.at[dgb], adg, asem)

        def _wait_gather(st, rg):
            pltpu.make_async_copy(tab_hbm.at[pl.ds(0, _EB)], rg[0],
                                  gsem).wait()
            pltpu.make_async_copy(as_hbm.at[pl.ds(0, _EB)], st[4],
                                  asem).wait()
            pltpu.make_async_copy(as_hbm.at[pl.ds(0, _EB)], st[5],
                                  asem).wait()

        def _wait_rows_scatter(rg):
            pltpu.make_async_copy(tab_hbm.at[pl.ds(0, _EB)], rg[0],
                                  ssem).wait()

        def _wait_den(rg):
            pltpu.make_async_copy(as_hbm.at[pl.ds(0, _EB)], rg[2],
                                  dsem).wait()

        def _process(b, st, rg, par):
            dsb, asg, adg = st[3], st[4], st[5]
            rows, dsc, exb = rg
            base = e0 + b * _EB
            for k in range(_KG):
                dsc[pl.ds(k * 16, 16)] = dsb[pl.ds(k * 16, 16)]
                z = asg[pl.ds(k * 16, 16)] + adg[pl.ds(k * 16, 16)]
                e = jnp.where(z >= 0, z, 0.2 * z)
                ex = jnp.exp(e - mv)
                eidx = base + k * 16 + lax.iota(jnp.int32, 16)
                ex = jnp.where(eidx < e_true, ex, 0.0)
                exb[pl.ds(k * 16, 16)] = ex

            @pl.when(c == par)
            def _():
                pltpu.async_copy(exb, den_sp.at[dsc], dsem, add=True)

            def _scale(i2, icarry):
                i = 2 * i2
                wv0 = plsc.load_gather(exb, [jnp.zeros((16,), jnp.int32) + i])
                wv1 = plsc.load_gather(exb,
                                       [jnp.zeros((16,), jnp.int32) + i + 1])
                for k in range(_H // 16):
                    rows[i, pl.ds(k * 16, 16)] = (
                        rows[i, pl.ds(k * 16, 16)] * wv0)
                for k in range(_H // 16):
                    rows[i + 1, pl.ds(k * 16, 16)] = (
                        rows[i + 1, pl.ds(k * 16, 16)] * wv1)
                return icarry

            lax.fori_loop(0, _EB // 2, _scale, 0)
            pltpu.async_copy(rows, acc_sp.at[dsc], ssem, add=True)

        def _half(b, cs, ns, cr, nr, pc):
            @pl.when(b + 1 < rpw)
            def _():
                _wait_idx(ns)

                @pl.when(b >= 2)
                def _():
                    _wait_rows_scatter(nr)

                    @pl.when(c == pc)
                    def _():
                        _wait_den(nr)

                _fire_gather(ns, nr)

            _wait_gather(cs, cr)
            _process(b, cs, cr, pc)

            @pl.when(b + 2 < rpw)
            def _():
                _fire_idx(b + 2, cs)

        _fire_idx(0, sets[0])
        _wait_idx(sets[0])
        _fire_gather(sets[0], rgs[0])
        _fire_idx(1, sets[1])

        def _outer(j6, carry):
            for t in range(6):
                _half(6 * j6 + t, sets[t % 2], sets[(t + 1) % 2],
                      rgs[t % 3], rgs[(t + 1) % 3], t % 2)
            return carry

        lax.fori_loop(0, rpw // 6, _outer, 0)
        _wait_rows_scatter(rgs[0])
        _wait_rows_scatter(rgs[1])
        _wait_rows_scatter(rgs[2])

        @pl.when(c == 1)
        def _():
            _wait_den(rgs[0])
            _wait_den(rgs[2])

        @pl.when(c == 0)
        def _():
            _wait_den(rgs[1])

        plsc.subcore_barrier()

        def _ocp(t, carry):
            rr = s * _TPB + t * 128
            pltpu.sync_copy(acc_sp.at[pl.ds(rr, 128)],
                            acc_hbm.at[c, pl.ds(rr, 128)])
            return carry

        lax.fori_loop(0, _TPB // 128, _ocp, 0)

        pltpu.sync_copy(den_sp.at[pl.ds(s * _TPB, _TPB)],
                        den_hbm.at[c, pl.ds(s * _TPB, _TPB)])

    return gat_pass


def kernel(x, edge_index, spatial_list, wnn_list, W1, a_src1, a_dst1, b1,
           W2, a_src2, a_dst2, b2, disc_W, disc_b):
    i32 = jnp.int32
    f32 = jnp.float32
    n = x.shape[0]

    loops = jnp.arange(n, dtype=i32)
    perm = jax.random.permutation(jax.random.key(42), n).astype(i32)

    blk = 16 * _EB * 6
    e1_true = edge_index.shape[1] + n
    ep1 = ((e1_true + blk - 1) // blk) * blk
    er_true = edge_index.shape[1]
    epr = ((er_true + blk - 1) // blk) * blk

    def pad1d(a, ep):
        a = a.astype(i32)
        return jnp.concatenate([a, jnp.zeros((ep - a.shape[0],), i32)])

    src_w = jnp.concatenate([wnn_list[0].astype(i32), loops])
    dst_w = jnp.concatenate([wnn_list[1].astype(i32), loops])
    src_s = jnp.concatenate([spatial_list[0].astype(i32), loops])
    dst_s = jnp.concatenate([spatial_list[1].astype(i32), loops])

    sgw1 = pad1d(src_w, ep1)
    dgw1 = pad1d(dst_w, ep1)
    sgw2 = pad1d(perm[src_w], ep1)
    dgw2 = pad1d(perm[dst_w], ep1)
    sgs = pad1d(src_s, ep1)
    dgs = pad1d(dst_s, ep1)
    sgr = pad1d(edge_index[1], epr)
    dsr = pad1d(edge_index[0], epr)

    xp = jnp.pad(x.astype(f32), ((0, _NP - n), (0, 0)))
    as1_2d = a_src1.reshape(_D, 1).astype(f32)
    ad1_2d = a_dst1.reshape(_D, 1).astype(f32)
    as2_2d = a_src2.reshape(_D, 1).astype(f32)
    ad2_2d = a_dst2.reshape(_D, 1).astype(f32)
    b1_2d = b1.reshape(1, _D).astype(f32)
    b2_2d = b2.reshape(1, _D).astype(f32)
    db_2d = disc_b.reshape(1, 1).astype(f32)

    def mvec(m):
        return jnp.full((16,), jnp.maximum(m[0, 0] + m[0, 1], 0.0), f32)

    gat_pass = _make_gat_pass(ep1, e1_true)
    readout_pass = _make_gat_pass(epr, er_true)

    h1t, as1, ad1, m1 = _run_k1(xp, W1.astype(f32), as1_2d, ad1_2d)
    tab1 = h1t.reshape(2 * _NP, _H)
    as1f = as1.reshape(_NP)
    ad1f = ad1.reshape(_NP)
    m1v = mvec(m1)

    acc11, den11 = gat_pass(as1f, ad1f, tab1, sgw1, dgw1, dgw1, m1v)
    acc12, den12 = gat_pass(as1f, ad1f, tab1, sgw2, dgw2, dgw1, m1v)

    h2t1, as21, ad21, m21 = _run_k2(acc11, den11.T, b1_2d,
                                    W2.astype(f32), as2_2d, ad2_2d)
    h2t2, as22, ad22, m22 = _run_k2(acc12, den12.T, b1_2d,
                                    W2.astype(f32), as2_2d, ad2_2d)

    acc21, den21 = gat_pass(as21.reshape(_NP), ad21.reshape(_NP),
                            h2t1.reshape(2 * _NP, _H), sgs, dgs, dgs,
                            mvec(m21))
    acc22, den22 = gat_pass(as22.reshape(_NP), ad22.reshape(_NP),
                            h2t2.reshape(2 * _NP, _H), sgs, dgs, dgs,
                            mvec(m22))

    h2_full, embt = _run_kemb(acc21, den21.T, b2_2d)

    zn = jnp.zeros((_NP,), f32)
    z16 = jnp.zeros((16,), f32)
    accr, denr = readout_pass(zn, zn, embt.reshape(2 * _NP, _H),
                              sgr, sgr, dsr, z16)

    ret_full = _run_k3(accr, denr.T, embt, acc22,
                       den22.T, b2_2d,
                       disc_W.astype(f32), db_2d)

    return h2_full[:n], ret_full[:n]
